# baseline (speedup 1.0000x reference)
import functools

import jax
import jax.numpy as jnp
from jax import lax
from jax.experimental import pallas as pl
from jax.experimental.pallas import tpu as pltpu
from jax.experimental.pallas import tpu_sc as plsc

N_EMBED = 1024
D_EMBED = 256
BETA = 0.25
N_TOK = 16 * 1024

BR = 512
NB = N_TOK // BR

NW = 32
B_PER_W = N_TOK // NW
CH = 128


def _argmax_body(z_ref, w_ref, idx_ref, loss_ref, wn_ref, wp_ref, acc_ref):
    i = pl.program_id(0)

    @pl.when(i == 0)
    def _():
        w = w_ref[...]
        n2 = jnp.sum(w * w, axis=1, keepdims=True)
        n1 = jnp.sqrt(n2)
        wn_ref[...] = w / jnp.maximum(n1, 1e-12)
        col = lax.broadcasted_iota(jnp.int32, (N_EMBED, 8), 1)
        j = lax.broadcasted_iota(jnp.int32, (N_EMBED, 8), 0)
        jhi = lax.shift_right_logical(j, 3).astype(jnp.float32)
        jlo = (j & 7).astype(jnp.float32)
        wp_ref[...] = jnp.where(
            col == 0, n2, jnp.where(
                col == 1, n1, jnp.where(
                    col == 2, jhi, jnp.where(
                        col == 3, jlo, jnp.where(
                            col == 4, 1.0, 0.0)))))
        acc_ref[0, 0] = 0.0

    zb = z_ref[...]
    zsq = zb * zb
    z2 = jnp.sum(zsq, axis=1, keepdims=True)
    znorm = jnp.maximum(jnp.sqrt(z2), 1e-12)
    zn = zb / znorm
    s = lax.dot_general(wn_ref[...], zn, (((1,), (1,)), ((), ())),
                        preferred_element_type=jnp.float32)
    m = jnp.max(s, axis=0, keepdims=True)
    hit = s == m
    oh = hit.astype(jnp.float32)
    g = lax.dot_general(wp_ref[...], oh, (((0,), (0,)), ((), ())),
                        preferred_element_type=jnp.float32)
    idxf = 8.0 * g[2:3, :] + g[3:4, :]
    idx_ref[...] = idxf.astype(jnp.int32).reshape(1, 1, BR)

    @pl.when(jnp.any(g[4:5, :] > 1.5))
    def _():
        jf = lax.broadcasted_iota(jnp.int32, s.shape, 0)
        idxe = jnp.max(jnp.where(hit, jf, -1), axis=0, keepdims=True)
        idx_ref[...] = idxe.reshape(1, 1, BR)

    z2l = lax.dot_general(jnp.ones((1, D_EMBED), jnp.float32), zsq,
                          (((1,), (1,)), ((), ())),
                          preferred_element_type=jnp.float32)
    znl = jnp.maximum(jnp.sqrt(z2l), 1e-12)
    loss_rows = g[0:1, :] - 2.0 * (m * znl) * g[1:2, :] + z2l
    acc_ref[0, 0] += jnp.sum(loss_rows)

    @pl.when(i == NB - 1)
    def _():
        mse = acc_ref[0, 0] / (N_TOK * D_EMBED)
        loss_ref[...] = jnp.full((1, 1), BETA * mse + mse, jnp.float32)


def _encode_body(idx_ref, enc_ref, perp_ref, cnt_ref):
    i = pl.program_id(0)

    @pl.when(i == 0)
    def _():
        cnt_ref[...] = jnp.zeros_like(cnt_ref)

    idx = idx_ref[0, 0, :]
    j = lax.broadcasted_iota(jnp.int32, (BR, N_EMBED), 1)
    oh = (j == idx[:, None]).astype(jnp.float32)
    enc_ref[...] = oh
    cnt_ref[...] += jnp.sum(oh, axis=0, keepdims=True)

    @pl.when(i == NB - 1)
    def _():
        p = cnt_ref[...] * (1.0 / N_TOK)
        perp = jnp.exp(-jnp.sum(p * jnp.log(p + 1e-10)))
        perp_ref[...] = jnp.full((1, 1), perp, jnp.float32)


def _sc_gather(W, idx):
    mesh = plsc.VectorSubcoreMesh(core_axis_name="c", subcore_axis_name="s")

    @functools.partial(
        pl.kernel, mesh=mesh,
        out_type=jax.ShapeDtypeStruct((N_TOK, D_EMBED), jnp.float32),
        scratch_types=[
            pltpu.VMEM((CH,), jnp.int32),
            pltpu.VMEM((CH, D_EMBED), jnp.float32),
            pltpu.SemaphoreType.DMA,
        ],
    )
    def gather_k(idx_hbm, w_hbm, out_hbm, idx_v, rows_v, sem):
        wid = lax.axis_index("s") * 2 + lax.axis_index("c")
        base = wid * B_PER_W
        for c in range(B_PER_W // CH):
            off = base + c * CH
            pltpu.sync_copy(idx_hbm.at[pl.ds(off, CH)], idx_v)
            pltpu.async_copy(w_hbm.at[idx_v], rows_v, sem).wait()
            pltpu.sync_copy(rows_v, out_hbm.at[pl.ds(off, CH)])

    return gather_k(idx, W)


def kernel(z, W):
    z_flat = z.reshape(N_TOK, D_EMBED)

    idx3, loss = pl.pallas_call(
        _argmax_body,
        grid=(NB,),
        in_specs=[
            pl.BlockSpec((BR, D_EMBED), lambda i: (i, 0)),
            pl.BlockSpec((N_EMBED, D_EMBED), lambda i: (0, 0)),
        ],
        out_specs=[
            pl.BlockSpec((1, 1, BR), lambda i: (i, 0, 0)),
            pl.BlockSpec((1, 1), lambda i: (0, 0)),
        ],
        out_shape=[
            jax.ShapeDtypeStruct((NB, 1, BR), jnp.int32),
            jax.ShapeDtypeStruct((1, 1), jnp.float32),
        ],
        scratch_shapes=[
            pltpu.VMEM((N_EMBED, D_EMBED), jnp.float32),
            pltpu.VMEM((N_EMBED, 8), jnp.float32),
            pltpu.SMEM((1, 1), jnp.float32),
        ],
    )(z_flat, W)

    idx = idx3.reshape(N_TOK)
    zq_flat = _sc_gather(W, idx)

    enc, perp = pl.pallas_call(
        _encode_body,
        grid=(NB,),
        in_specs=[
            pl.BlockSpec((1, 1, BR), lambda i: (i, 0, 0)),
        ],
        out_specs=[
            pl.BlockSpec((BR, N_EMBED), lambda i: (i, 0)),
            pl.BlockSpec((1, 1), lambda i: (0, 0)),
        ],
        out_shape=[
            jax.ShapeDtypeStruct((N_TOK, N_EMBED), jnp.float32),
            jax.ShapeDtypeStruct((1, 1), jnp.float32),
        ],
        scratch_shapes=[
            pltpu.VMEM((1, N_EMBED), jnp.float32),
        ],
    )(idx3)

    z_q = zq_flat.reshape(z.shape)
    return (z_q, loss.reshape(()), perp.reshape(()), enc, idx)


# speedup vs baseline: 1.2388x; 1.2388x over previous
import functools

import jax
import jax.numpy as jnp
from jax import lax
from jax.experimental import pallas as pl
from jax.experimental.pallas import tpu as pltpu
from jax.experimental.pallas import tpu_sc as plsc

N_EMBED = 1024
D_EMBED = 256
BETA = 0.25
N_TOK = 16 * 1024

BR = 1024
NB = N_TOK // BR

NW = 32
B_PER_W = N_TOK // NW
CH = 128


def _argmax_body(z_ref, w_ref, idx_ref, loss_ref, wn_ref, wp_ref, acc_ref):
    i = pl.program_id(0)

    @pl.when(i == 0)
    def _():
        w = w_ref[...]
        n2 = jnp.sum(w * w, axis=1, keepdims=True)
        n1 = jnp.sqrt(n2)
        wn_ref[...] = w / jnp.maximum(n1, 1e-12)
        col = lax.broadcasted_iota(jnp.int32, (N_EMBED, 8), 1)
        j = lax.broadcasted_iota(jnp.int32, (N_EMBED, 8), 0)
        jhi = lax.shift_right_logical(j, 3).astype(jnp.float32)
        jlo = (j & 7).astype(jnp.float32)
        wp_ref[...] = jnp.where(
            col == 0, n2, jnp.where(
                col == 1, n1, jnp.where(
                    col == 2, jhi, jnp.where(
                        col == 3, jlo, jnp.where(
                            col == 4, 1.0, 0.0)))))
        acc_ref[0, 0] = 0.0

    zb = z_ref[...]
    zsq = zb * zb
    z2 = jnp.sum(zsq, axis=1, keepdims=True)
    znorm = jnp.maximum(jnp.sqrt(z2), 1e-12)
    zn = zb / znorm
    s = lax.dot_general(wn_ref[...], zn, (((1,), (1,)), ((), ())),
                        preferred_element_type=jnp.float32)
    m = jnp.max(s, axis=0, keepdims=True)
    hit = s == m
    oh = hit.astype(jnp.float32)
    g = lax.dot_general(wp_ref[...], oh, (((0,), (0,)), ((), ())),
                        preferred_element_type=jnp.float32)
    idxf = 8.0 * g[2:3, :] + g[3:4, :]
    idx_ref[...] = idxf.astype(jnp.int32).reshape(1, 1, BR)

    @pl.when(jnp.any(g[4:5, :] > 1.5))
    def _():
        jf = lax.broadcasted_iota(jnp.int32, s.shape, 0)
        idxe = jnp.max(jnp.where(hit, jf, -1), axis=0, keepdims=True)
        idx_ref[...] = idxe.reshape(1, 1, BR)

    z2l = lax.dot_general(jnp.ones((1, D_EMBED), jnp.float32), zsq,
                          (((1,), (1,)), ((), ())),
                          preferred_element_type=jnp.float32)
    znl = jnp.maximum(jnp.sqrt(z2l), 1e-12)
    loss_rows = g[0:1, :] - 2.0 * (m * znl) * g[1:2, :] + z2l
    acc_ref[0, 0] += jnp.sum(loss_rows)

    @pl.when(i == NB - 1)
    def _():
        mse = acc_ref[0, 0] / (N_TOK * D_EMBED)
        loss_ref[...] = jnp.full((1, 1), BETA * mse + mse, jnp.float32)


def _encode_body(idx_ref, enc_ref, perp_ref, cnt_ref):
    i = pl.program_id(0)

    @pl.when(i == 0)
    def _():
        cnt_ref[...] = jnp.zeros_like(cnt_ref)

    idx = idx_ref[0, 0, :]
    j = lax.broadcasted_iota(jnp.int32, (BR, N_EMBED), 1)
    oh = (j == idx[:, None]).astype(jnp.float32)
    enc_ref[...] = oh
    cnt_ref[...] += jnp.sum(oh, axis=0, keepdims=True)

    @pl.when(i == NB - 1)
    def _():
        p = cnt_ref[...] * (1.0 / N_TOK)
        perp = jnp.exp(-jnp.sum(p * jnp.log(p + 1e-10)))
        perp_ref[...] = jnp.full((1, 1), perp, jnp.float32)


def _sc_gather(W, idx):
    mesh = plsc.VectorSubcoreMesh(core_axis_name="c", subcore_axis_name="s")

    @functools.partial(
        pl.kernel, mesh=mesh,
        out_type=jax.ShapeDtypeStruct((N_TOK, D_EMBED), jnp.float32),
        scratch_types=[
            pltpu.VMEM((CH,), jnp.int32),
            pltpu.VMEM((CH, D_EMBED), jnp.float32),
            pltpu.SemaphoreType.DMA,
        ],
    )
    def gather_k(idx_hbm, w_hbm, out_hbm, idx_v, rows_v, sem):
        wid = lax.axis_index("s") * 2 + lax.axis_index("c")
        base = wid * B_PER_W
        for c in range(B_PER_W // CH):
            off = base + c * CH
            pltpu.sync_copy(idx_hbm.at[pl.ds(off, CH)], idx_v)
            pltpu.async_copy(w_hbm.at[idx_v], rows_v, sem).wait()
            pltpu.sync_copy(rows_v, out_hbm.at[pl.ds(off, CH)])

    return gather_k(idx, W)


def kernel(z, W):
    z_flat = z.reshape(N_TOK, D_EMBED)

    idx3, loss = pl.pallas_call(
        _argmax_body,
        grid=(NB,),
        in_specs=[
            pl.BlockSpec((BR, D_EMBED), lambda i: (i, 0)),
            pl.BlockSpec((N_EMBED, D_EMBED), lambda i: (0, 0)),
        ],
        out_specs=[
            pl.BlockSpec((1, 1, BR), lambda i: (i, 0, 0)),
            pl.BlockSpec((1, 1), lambda i: (0, 0)),
        ],
        out_shape=[
            jax.ShapeDtypeStruct((NB, 1, BR), jnp.int32),
            jax.ShapeDtypeStruct((1, 1), jnp.float32),
        ],
        scratch_shapes=[
            pltpu.VMEM((N_EMBED, D_EMBED), jnp.float32),
            pltpu.VMEM((N_EMBED, 8), jnp.float32),
            pltpu.SMEM((1, 1), jnp.float32),
        ],
    )(z_flat, W)

    idx = idx3.reshape(N_TOK)
    zq_flat = _sc_gather(W, idx)

    enc, perp = pl.pallas_call(
        _encode_body,
        grid=(NB,),
        in_specs=[
            pl.BlockSpec((1, 1, BR), lambda i: (i, 0, 0)),
        ],
        out_specs=[
            pl.BlockSpec((BR, N_EMBED), lambda i: (i, 0)),
            pl.BlockSpec((1, 1), lambda i: (0, 0)),
        ],
        out_shape=[
            jax.ShapeDtypeStruct((N_TOK, N_EMBED), jnp.float32),
            jax.ShapeDtypeStruct((1, 1), jnp.float32),
        ],
        scratch_shapes=[
            pltpu.VMEM((1, N_EMBED), jnp.float32),
        ],
    )(idx3)

    z_q = zq_flat.reshape(z.shape)
    return (z_q, loss.reshape(()), perp.reshape(()), enc, idx)


# speedup vs baseline: 1.3743x; 1.1094x over previous
import functools

import jax
import jax.numpy as jnp
from jax import lax
from jax.experimental import pallas as pl
from jax.experimental.pallas import tpu as pltpu
from jax.experimental.pallas import tpu_sc as plsc

N_EMBED = 1024
D_EMBED = 256
BETA = 0.25
N_TOK = 16 * 1024

BR = 2048
NB = N_TOK // BR

NW = 32
B_PER_W = N_TOK // NW
CH = 128


def _argmax_body(z_ref, w_ref, idx_ref, loss_ref, wn_ref, wp_ref, acc_ref):
    i = pl.program_id(0)

    @pl.when(i == 0)
    def _():
        w = w_ref[...]
        n2 = jnp.sum(w * w, axis=1, keepdims=True)
        n1 = jnp.sqrt(n2)
        wn_ref[...] = w / jnp.maximum(n1, 1e-12)
        col = lax.broadcasted_iota(jnp.int32, (N_EMBED, 8), 1)
        j = lax.broadcasted_iota(jnp.int32, (N_EMBED, 8), 0)
        jhi = lax.shift_right_logical(j, 3).astype(jnp.float32)
        jlo = (j & 7).astype(jnp.float32)
        wp_ref[...] = jnp.where(
            col == 0, n2, jnp.where(
                col == 1, n1, jnp.where(
                    col == 2, jhi, jnp.where(
                        col == 3, jlo, jnp.where(
                            col == 4, 1.0, 0.0)))))
        acc_ref[0, 0] = 0.0

    zb = z_ref[...]
    zsq = zb * zb
    z2 = jnp.sum(zsq, axis=1, keepdims=True)
    znorm = jnp.maximum(jnp.sqrt(z2), 1e-12)
    zn = zb / znorm
    s = lax.dot_general(wn_ref[...], zn, (((1,), (1,)), ((), ())),
                        preferred_element_type=jnp.float32)
    m = jnp.max(s, axis=0, keepdims=True)
    hit = s == m
    oh = hit.astype(jnp.float32)
    g = lax.dot_general(wp_ref[...], oh, (((0,), (0,)), ((), ())),
                        preferred_element_type=jnp.float32)
    idxf = 8.0 * g[2:3, :] + g[3:4, :]
    idx_ref[...] = idxf.astype(jnp.int32).reshape(1, 1, BR)

    @pl.when(jnp.any(g[4:5, :] > 1.5))
    def _():
        jf = lax.broadcasted_iota(jnp.int32, s.shape, 0)
        idxe = jnp.max(jnp.where(hit, jf, -1), axis=0, keepdims=True)
        idx_ref[...] = idxe.reshape(1, 1, BR)

    z2l = lax.dot_general(jnp.ones((1, D_EMBED), jnp.float32), zsq,
                          (((1,), (1,)), ((), ())),
                          preferred_element_type=jnp.float32)
    znl = jnp.maximum(jnp.sqrt(z2l), 1e-12)
    loss_rows = g[0:1, :] - 2.0 * (m * znl) * g[1:2, :] + z2l
    acc_ref[0, 0] += jnp.sum(loss_rows)

    @pl.when(i == NB - 1)
    def _():
        mse = acc_ref[0, 0] / (N_TOK * D_EMBED)
        loss_ref[...] = jnp.full((1, 1), BETA * mse + mse, jnp.float32)


def _encode_body(idx_ref, enc_ref, perp_ref, cnt_ref):
    i = pl.program_id(0)

    @pl.when(i == 0)
    def _():
        cnt_ref[...] = jnp.zeros_like(cnt_ref)

    idx = idx_ref[0, 0, :]
    j = lax.broadcasted_iota(jnp.int32, (BR, N_EMBED), 1)
    oh = (j == idx[:, None]).astype(jnp.float32)
    enc_ref[...] = oh
    cnt_ref[...] += jnp.sum(oh, axis=0, keepdims=True)

    @pl.when(i == NB - 1)
    def _():
        p = cnt_ref[...] * (1.0 / N_TOK)
        perp = jnp.exp(-jnp.sum(p * jnp.log(p + 1e-10)))
        perp_ref[...] = jnp.full((1, 1), perp, jnp.float32)


def _sc_gather(W, idx):
    mesh = plsc.VectorSubcoreMesh(core_axis_name="c", subcore_axis_name="s")

    @functools.partial(
        pl.kernel, mesh=mesh,
        out_type=jax.ShapeDtypeStruct((N_TOK, D_EMBED), jnp.float32),
        scratch_types=[
            pltpu.VMEM((CH,), jnp.int32),
            pltpu.VMEM((CH, D_EMBED), jnp.float32),
            pltpu.SemaphoreType.DMA,
        ],
    )
    def gather_k(idx_hbm, w_hbm, out_hbm, idx_v, rows_v, sem):
        wid = lax.axis_index("s") * 2 + lax.axis_index("c")
        base = wid * B_PER_W
        for c in range(B_PER_W // CH):
            off = base + c * CH
            pltpu.sync_copy(idx_hbm.at[pl.ds(off, CH)], idx_v)
            pltpu.async_copy(w_hbm.at[idx_v], rows_v, sem).wait()
            pltpu.sync_copy(rows_v, out_hbm.at[pl.ds(off, CH)])

    return gather_k(idx, W)


def kernel(z, W):
    z_flat = z.reshape(N_TOK, D_EMBED)

    idx3, loss = pl.pallas_call(
        _argmax_body,
        grid=(NB,),
        in_specs=[
            pl.BlockSpec((BR, D_EMBED), lambda i: (i, 0)),
            pl.BlockSpec((N_EMBED, D_EMBED), lambda i: (0, 0)),
        ],
        out_specs=[
            pl.BlockSpec((1, 1, BR), lambda i: (i, 0, 0)),
            pl.BlockSpec((1, 1), lambda i: (0, 0)),
        ],
        out_shape=[
            jax.ShapeDtypeStruct((NB, 1, BR), jnp.int32),
            jax.ShapeDtypeStruct((1, 1), jnp.float32),
        ],
        scratch_shapes=[
            pltpu.VMEM((N_EMBED, D_EMBED), jnp.float32),
            pltpu.VMEM((N_EMBED, 8), jnp.float32),
            pltpu.SMEM((1, 1), jnp.float32),
        ],
    )(z_flat, W)

    idx = idx3.reshape(N_TOK)
    zq_flat = _sc_gather(W, idx)

    enc, perp = pl.pallas_call(
        _encode_body,
        grid=(NB,),
        in_specs=[
            pl.BlockSpec((1, 1, BR), lambda i: (i, 0, 0)),
        ],
        out_specs=[
            pl.BlockSpec((BR, N_EMBED), lambda i: (i, 0)),
            pl.BlockSpec((1, 1), lambda i: (0, 0)),
        ],
        out_shape=[
            jax.ShapeDtypeStruct((N_TOK, N_EMBED), jnp.float32),
            jax.ShapeDtypeStruct((1, 1), jnp.float32),
        ],
        scratch_shapes=[
            pltpu.VMEM((1, N_EMBED), jnp.float32),
        ],
    )(idx3)

    z_q = zq_flat.reshape(z.shape)
    return (z_q, loss.reshape(()), perp.reshape(()), enc, idx)


# speedup vs baseline: 1.4055x; 1.0227x over previous
import functools

import jax
import jax.numpy as jnp
from jax import lax
from jax.experimental import pallas as pl
from jax.experimental.pallas import tpu as pltpu
from jax.experimental.pallas import tpu_sc as plsc

N_EMBED = 1024
D_EMBED = 256
BETA = 0.25
N_TOK = 16 * 1024

BR = 4096
NB = N_TOK // BR

NW = 32
B_PER_W = N_TOK // NW
CH = 128


def _argmax_body(z_ref, w_ref, idx_ref, loss_ref, wn_ref, wp_ref, acc_ref):
    i = pl.program_id(0)

    @pl.when(i == 0)
    def _():
        w = w_ref[...]
        n2 = jnp.sum(w * w, axis=1, keepdims=True)
        n1 = jnp.sqrt(n2)
        wn_ref[...] = w / jnp.maximum(n1, 1e-12)
        col = lax.broadcasted_iota(jnp.int32, (N_EMBED, 8), 1)
        j = lax.broadcasted_iota(jnp.int32, (N_EMBED, 8), 0)
        jhi = lax.shift_right_logical(j, 3).astype(jnp.float32)
        jlo = (j & 7).astype(jnp.float32)
        wp_ref[...] = jnp.where(
            col == 0, n2, jnp.where(
                col == 1, n1, jnp.where(
                    col == 2, jhi, jnp.where(
                        col == 3, jlo, jnp.where(
                            col == 4, 1.0, 0.0)))))
        acc_ref[0, 0] = 0.0

    zb = z_ref[...]
    zsq = zb * zb
    z2 = jnp.sum(zsq, axis=1, keepdims=True)
    znorm = jnp.maximum(jnp.sqrt(z2), 1e-12)
    zn = zb / znorm
    s = lax.dot_general(wn_ref[...], zn, (((1,), (1,)), ((), ())),
                        preferred_element_type=jnp.float32)
    m = jnp.max(s, axis=0, keepdims=True)
    hit = s == m
    oh = hit.astype(jnp.float32)
    g = lax.dot_general(wp_ref[...], oh, (((0,), (0,)), ((), ())),
                        preferred_element_type=jnp.float32)
    idxf = 8.0 * g[2:3, :] + g[3:4, :]
    idx_ref[...] = idxf.astype(jnp.int32).reshape(1, 1, BR)

    @pl.when(jnp.any(g[4:5, :] > 1.5))
    def _():
        jf = lax.broadcasted_iota(jnp.int32, s.shape, 0)
        idxe = jnp.max(jnp.where(hit, jf, -1), axis=0, keepdims=True)
        idx_ref[...] = idxe.reshape(1, 1, BR)

    z2l = lax.dot_general(jnp.ones((1, D_EMBED), jnp.float32), zsq,
                          (((1,), (1,)), ((), ())),
                          preferred_element_type=jnp.float32)
    znl = jnp.maximum(jnp.sqrt(z2l), 1e-12)
    loss_rows = g[0:1, :] - 2.0 * (m * znl) * g[1:2, :] + z2l
    acc_ref[0, 0] += jnp.sum(loss_rows)

    @pl.when(i == NB - 1)
    def _():
        mse = acc_ref[0, 0] / (N_TOK * D_EMBED)
        loss_ref[...] = jnp.full((1, 1), BETA * mse + mse, jnp.float32)


def _encode_body(idx_ref, enc_ref, perp_ref, cnt_ref):
    i = pl.program_id(0)

    @pl.when(i == 0)
    def _():
        cnt_ref[...] = jnp.zeros_like(cnt_ref)

    idx = idx_ref[0, 0, :]
    j = lax.broadcasted_iota(jnp.int32, (BR, N_EMBED), 1)
    oh = (j == idx[:, None]).astype(jnp.float32)
    enc_ref[...] = oh
    cnt_ref[...] += jnp.sum(oh, axis=0, keepdims=True)

    @pl.when(i == NB - 1)
    def _():
        p = cnt_ref[...] * (1.0 / N_TOK)
        perp = jnp.exp(-jnp.sum(p * jnp.log(p + 1e-10)))
        perp_ref[...] = jnp.full((1, 1), perp, jnp.float32)


def _sc_gather(W, idx):
    mesh = plsc.VectorSubcoreMesh(core_axis_name="c", subcore_axis_name="s")

    @functools.partial(
        pl.kernel, mesh=mesh,
        out_type=jax.ShapeDtypeStruct((N_TOK, D_EMBED), jnp.float32),
        scratch_types=[
            pltpu.VMEM((CH,), jnp.int32),
            pltpu.VMEM((CH, D_EMBED), jnp.float32),
            pltpu.SemaphoreType.DMA,
        ],
    )
    def gather_k(idx_hbm, w_hbm, out_hbm, idx_v, rows_v, sem):
        wid = lax.axis_index("s") * 2 + lax.axis_index("c")
        base = wid * B_PER_W
        for c in range(B_PER_W // CH):
            off = base + c * CH
            pltpu.sync_copy(idx_hbm.at[pl.ds(off, CH)], idx_v)
            pltpu.async_copy(w_hbm.at[idx_v], rows_v, sem).wait()
            pltpu.sync_copy(rows_v, out_hbm.at[pl.ds(off, CH)])

    return gather_k(idx, W)


def kernel(z, W):
    z_flat = z.reshape(N_TOK, D_EMBED)

    idx3, loss = pl.pallas_call(
        _argmax_body,
        grid=(NB,),
        in_specs=[
            pl.BlockSpec((BR, D_EMBED), lambda i: (i, 0)),
            pl.BlockSpec((N_EMBED, D_EMBED), lambda i: (0, 0)),
        ],
        out_specs=[
            pl.BlockSpec((1, 1, BR), lambda i: (i, 0, 0)),
            pl.BlockSpec((1, 1), lambda i: (0, 0)),
        ],
        out_shape=[
            jax.ShapeDtypeStruct((NB, 1, BR), jnp.int32),
            jax.ShapeDtypeStruct((1, 1), jnp.float32),
        ],
        scratch_shapes=[
            pltpu.VMEM((N_EMBED, D_EMBED), jnp.float32),
            pltpu.VMEM((N_EMBED, 8), jnp.float32),
            pltpu.SMEM((1, 1), jnp.float32),
        ],
    )(z_flat, W)

    idx = idx3.reshape(N_TOK)
    zq_flat = _sc_gather(W, idx)

    enc, perp = pl.pallas_call(
        _encode_body,
        grid=(NB,),
        in_specs=[
            pl.BlockSpec((1, 1, BR), lambda i: (i, 0, 0)),
        ],
        out_specs=[
            pl.BlockSpec((BR, N_EMBED), lambda i: (i, 0)),
            pl.BlockSpec((1, 1), lambda i: (0, 0)),
        ],
        out_shape=[
            jax.ShapeDtypeStruct((N_TOK, N_EMBED), jnp.float32),
            jax.ShapeDtypeStruct((1, 1), jnp.float32),
        ],
        scratch_shapes=[
            pltpu.VMEM((1, N_EMBED), jnp.float32),
        ],
    )(idx3)

    z_q = zq_flat.reshape(z.shape)
    return (z_q, loss.reshape(()), perp.reshape(()), enc, idx)


# speedup vs baseline: 1.4080x; 1.0018x over previous
import functools

import jax
import jax.numpy as jnp
from jax import lax
from jax.experimental import pallas as pl
from jax.experimental.pallas import tpu as pltpu
from jax.experimental.pallas import tpu_sc as plsc

N_EMBED = 1024
D_EMBED = 256
BETA = 0.25
N_TOK = 16 * 1024

BR = 4096
NB = N_TOK // BR

NW = 32
B_PER_W = N_TOK // NW
CH = 128


def _argmax_body(z_ref, w_ref, idx_ref, loss_ref, wn_ref, wp_ref, acc_ref):
    i = pl.program_id(0)

    @pl.when(i == 0)
    def _():
        w = w_ref[...]
        n2 = jnp.sum(w * w, axis=1, keepdims=True)
        n1 = jnp.sqrt(n2)
        wn_ref[...] = w / jnp.maximum(n1, 1e-12)
        col = lax.broadcasted_iota(jnp.int32, (N_EMBED, 8), 1)
        j = lax.broadcasted_iota(jnp.int32, (N_EMBED, 8), 0)
        jhi = lax.shift_right_logical(j, 3).astype(jnp.float32)
        jlo = (j & 7).astype(jnp.float32)
        wp_ref[...] = jnp.where(
            col == 0, n2, jnp.where(
                col == 1, n1, jnp.where(
                    col == 2, jhi, jnp.where(
                        col == 3, jlo, jnp.where(
                            col == 4, 1.0, 0.0)))))
        acc_ref[0, 0] = 0.0

    zb = z_ref[...]
    zsq = zb * zb
    z2 = jnp.sum(zsq, axis=1, keepdims=True)
    znorm = jnp.maximum(jnp.sqrt(z2), 1e-12)
    zn = zb / znorm
    s = lax.dot_general(wn_ref[...], zn, (((1,), (1,)), ((), ())),
                        preferred_element_type=jnp.float32)
    m = jnp.max(s, axis=0, keepdims=True)
    hit = s == m
    oh = hit.astype(jnp.float32)
    g = lax.dot_general(wp_ref[...], oh, (((0,), (0,)), ((), ())),
                        preferred_element_type=jnp.float32)
    idxf = 8.0 * g[2:3, :] + g[3:4, :]
    idx_ref[...] = idxf.astype(jnp.int32).reshape(1, 1, BR)

    @pl.when(jnp.any(g[4:5, :] > 1.5))
    def _():
        jf = lax.broadcasted_iota(jnp.int32, s.shape, 0)
        idxe = jnp.max(jnp.where(hit, jf, -1), axis=0, keepdims=True)
        idx_ref[...] = idxe.reshape(1, 1, BR)

    z2l = lax.dot_general(jnp.ones((1, D_EMBED), jnp.float32), zsq,
                          (((1,), (1,)), ((), ())),
                          preferred_element_type=jnp.float32)
    znl = jnp.maximum(jnp.sqrt(z2l), 1e-12)
    loss_rows = g[0:1, :] - 2.0 * (m * znl) * g[1:2, :] + z2l
    acc_ref[0, 0] += jnp.sum(loss_rows)

    @pl.when(i == NB - 1)
    def _():
        mse = acc_ref[0, 0] / (N_TOK * D_EMBED)
        loss_ref[...] = jnp.full((1, 1), BETA * mse + mse, jnp.float32)


def _encode_body(idx_ref, enc_ref, perp_ref, cnt_ref):
    i = pl.program_id(0)

    @pl.when(i == 0)
    def _():
        cnt_ref[...] = jnp.zeros_like(cnt_ref)

    idx = idx_ref[0, 0, :]
    j = lax.broadcasted_iota(jnp.int32, (BR, N_EMBED), 1)
    oh = (j == idx[:, None]).astype(jnp.float32)
    enc_ref[...] = oh
    cnt_ref[...] += jnp.sum(oh, axis=0, keepdims=True)

    @pl.when(i == NB - 1)
    def _():
        p = cnt_ref[...] * (1.0 / N_TOK)
        perp = jnp.exp(-jnp.sum(p * jnp.log(p + 1e-10)))
        perp_ref[...] = jnp.full((1, 1), perp, jnp.float32)


def _sc_gather(W, idx):
    mesh = plsc.VectorSubcoreMesh(core_axis_name="c", subcore_axis_name="s")

    @functools.partial(
        pl.kernel, mesh=mesh,
        out_type=jax.ShapeDtypeStruct((N_TOK, D_EMBED), jnp.float32),
        scratch_types=[
            pltpu.VMEM((B_PER_W,), jnp.int32),
            pltpu.VMEM((CH, D_EMBED), jnp.float32),
            pltpu.VMEM((CH, D_EMBED), jnp.float32),
            pltpu.SemaphoreType.DMA,
            pltpu.SemaphoreType.DMA,
            pltpu.SemaphoreType.DMA,
            pltpu.SemaphoreType.DMA,
        ],
    )
    def gather_k(idx_hbm, w_hbm, out_hbm, idx_v, rows0, rows1,
                 gsem0, gsem1, osem0, osem1):
        wid = lax.axis_index("s") * 2 + lax.axis_index("c")
        base = wid * B_PER_W
        rows = (rows0, rows1)
        gsem = (gsem0, gsem1)
        osem = (osem0, osem1)
        nch = B_PER_W // CH
        pltpu.sync_copy(idx_hbm.at[pl.ds(base, B_PER_W)], idx_v)
        gathers = [None] * nch
        outs = [None] * nch
        gathers[0] = pltpu.async_copy(
            w_hbm.at[idx_v.at[pl.ds(0, CH)]], rows[0], gsem[0])
        for c in range(nch):
            b = c % 2
            gathers[c].wait()
            if c + 1 < nch:
                if c >= 1:
                    outs[c - 1].wait()
                gathers[c + 1] = pltpu.async_copy(
                    w_hbm.at[idx_v.at[pl.ds((c + 1) * CH, CH)]],
                    rows[1 - b], gsem[1 - b])
            outs[c] = pltpu.async_copy(
                rows[b], out_hbm.at[pl.ds(base + c * CH, CH)], osem[b])
        outs[nch - 2].wait()
        outs[nch - 1].wait()

    return gather_k(idx, W)


def kernel(z, W):
    z_flat = z.reshape(N_TOK, D_EMBED)

    idx3, loss = pl.pallas_call(
        _argmax_body,
        grid=(NB,),
        in_specs=[
            pl.BlockSpec((BR, D_EMBED), lambda i: (i, 0)),
            pl.BlockSpec((N_EMBED, D_EMBED), lambda i: (0, 0)),
        ],
        out_specs=[
            pl.BlockSpec((1, 1, BR), lambda i: (i, 0, 0)),
            pl.BlockSpec((1, 1), lambda i: (0, 0)),
        ],
        out_shape=[
            jax.ShapeDtypeStruct((NB, 1, BR), jnp.int32),
            jax.ShapeDtypeStruct((1, 1), jnp.float32),
        ],
        scratch_shapes=[
            pltpu.VMEM((N_EMBED, D_EMBED), jnp.float32),
            pltpu.VMEM((N_EMBED, 8), jnp.float32),
            pltpu.SMEM((1, 1), jnp.float32),
        ],
    )(z_flat, W)

    idx = idx3.reshape(N_TOK)
    zq_flat = _sc_gather(W, idx)

    enc, perp = pl.pallas_call(
        _encode_body,
        grid=(NB,),
        in_specs=[
            pl.BlockSpec((1, 1, BR), lambda i: (i, 0, 0)),
        ],
        out_specs=[
            pl.BlockSpec((BR, N_EMBED), lambda i: (i, 0)),
            pl.BlockSpec((1, 1), lambda i: (0, 0)),
        ],
        out_shape=[
            jax.ShapeDtypeStruct((N_TOK, N_EMBED), jnp.float32),
            jax.ShapeDtypeStruct((1, 1), jnp.float32),
        ],
        scratch_shapes=[
            pltpu.VMEM((1, N_EMBED), jnp.float32),
        ],
    )(idx3)

    z_q = zq_flat.reshape(z.shape)
    return (z_q, loss.reshape(()), perp.reshape(()), enc, idx)
